# Initial kernel scaffold; baseline (speedup 1.0000x reference)
import jax
import jax.numpy as jnp
from jax.experimental import pallas as pl


def kernel(y_soft, edge_index, y_true, mask):
    raise NotImplementedError("write your pallas kernel here")


# speedup vs baseline: 35.5619x; 35.5619x over previous
import functools

import jax
import jax.numpy as jnp
from jax import lax
from jax.experimental import pallas as pl
from jax.experimental.pallas import tpu as pltpu
from jax.experimental.pallas import tpu_sc as plsc

N = 100000
E = 3200000
C = 16
NUM_CORRECTION_LAYERS = 10
CORRECTION_ALPHA = 0.979
NUM_SMOOTHING_LAYERS = 10
SMOOTHING_ALPHA = 0.756

NC = 2
NS = 16
NW = NC * NS

CHUNK = 128
CPB = 8
BLOCKS = 98
WCHUNKS = CPB * BLOCKS
WEDGES = WCHUNKS * CHUNK
EPAD = NW * WEDGES
NPAD = 100352
NDUMP = NPAD - N
ZROWS = NPAD // NS
ZCOPY = ZROWS // CHUNK
NOUT = 100096
OUTROWS = NOUT // NS
NBUF = 4


def _make_seg(gather: bool):
  mesh = plsc.VectorSubcoreMesh(core_axis_name="c", subcore_axis_name="s")
  scratch = [
      pltpu.VMEM((2, CPB, CHUNK), jnp.int32),
      pltpu.VMEM((2, CPB, CHUNK), jnp.int32),
      pltpu.VMEM((NBUF, CHUNK, C), jnp.float32),
      pltpu.VMEM_SHARED((NPAD, C), jnp.float32),
      pltpu.SemaphoreType.DMA,
      pltpu.SemaphoreType.DMA,
      pltpu.SemaphoreType.DMA,
      pltpu.SemaphoreType.DMA,
      pltpu.SemaphoreType.DMA,
      pltpu.SemaphoreType.DMA,
  ]

  def body(z_hbm, src_hbm, dst_hbm, out_hbm,
           idx_src, idx_dst, rows, acc, sg0, sg1, sg2, sg3, si0, si1):
    c = lax.axis_index("c")
    s = lax.axis_index("s")
    w = s * NC + c
    wrow = w * WCHUNKS
    sgs = (sg0, sg1, sg2, sg3)
    sis = (si0, si1)

    fill = jnp.zeros((C,), jnp.float32) if gather else jnp.ones((C,), jnp.float32)

    def fill_buf(b, val):
      def st(i, _):
        rows[b, i, :] = val
        return _
      lax.fori_loop(0, CHUNK, st, 0)

    fill_buf(0, jnp.zeros((C,), jnp.float32))
    zbase = s * ZROWS
    def zcp(i, _):
      pltpu.sync_copy(rows.at[0], acc.at[pl.ds(zbase + i * CHUNK, CHUNK)])
      return _
    lax.fori_loop(0, ZCOPY, zcp, 0)
    if not gather:
      for b in range(NBUF):
        fill_buf(b, fill)
    plsc.subcore_barrier()

    def issue_idx(blk, p):
      row = wrow + blk * CPB
      if gather:
        pltpu.async_copy(src_hbm.at[pl.ds(row, CPB)], idx_src.at[p], sis[p])
      pltpu.async_copy(dst_hbm.at[pl.ds(row, CPB)], idx_dst.at[p], sis[p])

    def wait_idx(p):
      if gather:
        pltpu.make_async_copy(src_hbm.at[pl.ds(0, CPB)], idx_src.at[p],
                              sis[p]).wait()
      pltpu.make_async_copy(dst_hbm.at[pl.ds(0, CPB)], idx_dst.at[p],
                            sis[p]).wait()

    issue_idx(0, 0)
    issue_idx(1, 1)

    def inner(p):
      def grp(i, _):
        cg = i * NBUF
        if gather:
          for b in range(NBUF):
            pltpu.async_copy(z_hbm.at[idx_src.at[p, cg + b]], rows.at[b],
                             sgs[b])
          for b in range(NBUF):
            pltpu.make_async_copy(z_hbm.at[idx_src.at[p, 0]], rows.at[b],
                                  sgs[b]).wait()
            pltpu.sync_copy(rows.at[b], acc.at[idx_dst.at[p, cg + b]],
                            add=True)
        else:
          for b in range(NBUF):
            pltpu.sync_copy(rows.at[b], acc.at[idx_dst.at[p, cg + b]],
                            add=True)
        return _
      lax.fori_loop(0, CPB // NBUF, grp, 0)

    def outer(i, _):
      for p in range(2):
        blk = i * 2 + p
        wait_idx(p)
        inner(p)
        nxt = blk + 2
        @pl.when(nxt < BLOCKS)
        def _issue():
          issue_idx(nxt, p)
      return _
    lax.fori_loop(0, BLOCKS // 2, outer, 0)

    plsc.subcore_barrier()
    obase = s * OUTROWS
    pltpu.sync_copy(acc.at[pl.ds(obase, OUTROWS)],
                    out_hbm.at[c, pl.ds(obase, OUTROWS)])

  return pl.kernel(
      body,
      out_type=jax.ShapeDtypeStruct((NC, NOUT, C), jnp.float32),
      mesh=mesh,
      scratch_types=scratch,
      compiler_params=pltpu.CompilerParams(use_tc_tiling_on_sc=False),
  )


def _seg_gather(z, src2, dst2):
  return _make_seg(gather=True)(z, src2, dst2)


def _seg_ones(z, src2, dst2):
  return _make_seg(gather=False)(z, src2, dst2)


def kernel(y_soft, edge_index, y_true, mask):
  src = edge_index[0]
  dst = edge_index[1]
  M = mask.shape[0]

  P = EPAD - E
  ar = jnp.arange(P, dtype=jnp.int32)
  pad_src = (ar * 97) % N
  pad_dst = N + (ar % NDUMP)
  src2 = jnp.concatenate([src, pad_src]).reshape(EPAD // CHUNK, CHUNK)
  dst2 = jnp.concatenate([dst, pad_dst]).reshape(EPAD // CHUNK, CHUNK)

  def seg(z):
    parts = _seg_gather(z, src2, dst2)
    return (parts[0] + parts[1])[:N]

  zdummy = jnp.zeros((N, C), jnp.float32)
  degp = _seg_ones(zdummy, src2, dst2)
  degs = jnp.clip((degp[0] + degp[1])[:N, 0], 1.0, None)
  norm = (degs ** -0.5)[:, None]

  y_true_oh = jax.nn.one_hot(y_true, C, dtype=y_soft.dtype)

  def label_prop(y, num_layers, alpha, lo, hi):
    last = (1.0 - alpha) * y
    for _ in range(num_layers):
      agg = seg(norm * y)
      y = jnp.clip(last + alpha * (agg * norm), lo, hi)
    return y

  err_top = y_true_oh - y_soft[:M]
  error = jnp.concatenate([err_top, jnp.zeros((N - M, C), y_soft.dtype)])
  smoothed = label_prop(error, NUM_CORRECTION_LAYERS, CORRECTION_ALPHA,
                        -1.0, 1.0)
  sigma = jnp.abs(err_top).sum() / M
  scale = sigma / jnp.abs(smoothed).sum(axis=1, keepdims=True)
  scale = jnp.where(jnp.isinf(scale) | (scale > 1000.0), 1.0, scale)
  result = y_soft + scale * smoothed
  result = jnp.where(jnp.isnan(result), y_soft, result)

  y = jnp.concatenate([y_true_oh, result[M:]])
  out = label_prop(y, NUM_SMOOTHING_LAYERS, SMOOTHING_ALPHA, 0.0, 1.0)
  return out


# speedup vs baseline: 46.1300x; 1.2972x over previous
import functools

import jax
import jax.numpy as jnp
from jax import lax
from jax.experimental import pallas as pl
from jax.experimental.pallas import tpu as pltpu
from jax.experimental.pallas import tpu_sc as plsc

N = 100000
E = 3200000
C = 16
NUM_CORRECTION_LAYERS = 10
CORRECTION_ALPHA = 0.979
NUM_SMOOTHING_LAYERS = 10
SMOOTHING_ALPHA = 0.756

NC = 2
NS = 16
NW = NC * NS

CHUNK = 128
CPB = 8
BLOCKS = 98
WCHUNKS = CPB * BLOCKS
WEDGES = WCHUNKS * CHUNK
EPAD = NW * WEDGES
NPAD = 100352
NDUMP = NPAD - N
ZROWS = NPAD // NS
ZCOPY = ZROWS // CHUNK
NOUT = 100096
OUTROWS = NOUT // NS
NBUF = 8


def _make_seg(gather: bool):
  mesh = plsc.VectorSubcoreMesh(core_axis_name="c", subcore_axis_name="s")
  scratch = (
      [pltpu.VMEM((2, CPB, CHUNK), jnp.int32),
       pltpu.VMEM((2, CPB, CHUNK), jnp.int32),
       pltpu.VMEM((NBUF, CHUNK, C), jnp.float32),
       pltpu.VMEM_SHARED((NPAD, C), jnp.float32)]
      + [pltpu.SemaphoreType.DMA] * (2 * NBUF + 2)
  )

  def body(z_hbm, src_hbm, dst_hbm, out_hbm,
           idx_src, idx_dst, rows, acc, *sems):
    c = lax.axis_index("c")
    s = lax.axis_index("s")
    w = s * NC + c
    wrow = w * WCHUNKS
    sg = sems[0:NBUF]
    ss = sems[NBUF:2 * NBUF]
    sis = sems[2 * NBUF:]

    def fill_buf(b, val):
      def st(i, _):
        rows[b, i, :] = val
        return _
      lax.fori_loop(0, CHUNK, st, 0)

    fill_buf(0, jnp.zeros((C,), jnp.float32))
    zbase = s * ZROWS
    def zcp(i, _):
      pltpu.sync_copy(rows.at[0], acc.at[pl.ds(zbase + i * CHUNK, CHUNK)])
      return _
    lax.fori_loop(0, ZCOPY, zcp, 0)
    if not gather:
      for b in range(NBUF):
        fill_buf(b, jnp.ones((C,), jnp.float32))
    plsc.subcore_barrier()

    def issue_idx(blk, p):
      row = wrow + blk * CPB
      if gather:
        pltpu.async_copy(src_hbm.at[pl.ds(row, CPB)], idx_src.at[p], sis[p])
      pltpu.async_copy(dst_hbm.at[pl.ds(row, CPB)], idx_dst.at[p], sis[p])

    def wait_idx(p):
      if gather:
        pltpu.make_async_copy(src_hbm.at[pl.ds(0, CPB)], idx_src.at[p],
                              sis[p]).wait()
      pltpu.make_async_copy(dst_hbm.at[pl.ds(0, CPB)], idx_dst.at[p],
                            sis[p]).wait()

    issue_idx(0, 0)
    issue_idx(1, 1)
    wait_idx(0)
    if gather:
      for b in range(NBUF):
        pltpu.async_copy(z_hbm.at[idx_src.at[0, b]], rows.at[b], sg[b])

    def rnd(r, _):
      p = r % 2
      q = 1 - p
      for b in range(NBUF):
        if gather:
          pltpu.make_async_copy(z_hbm.at[idx_src.at[0, 0]], rows.at[b],
                                sg[b]).wait()
        pltpu.async_copy(rows.at[b], acc.at[idx_dst.at[p, b]], ss[b],
                         add=True)
      @pl.when((r < BLOCKS - 1) & (q == 0))
      def _wi0():
        wait_idx(0)
      @pl.when((r < BLOCKS - 1) & (q == 1))
      def _wi1():
        wait_idx(1)
      for b in range(NBUF):
        pltpu.make_async_copy(rows.at[b], acc.at[idx_dst.at[0, 0]],
                              ss[b]).wait()
        if gather:
          @pl.when(r < BLOCKS - 1)
          def _fg():
            pltpu.async_copy(z_hbm.at[idx_src.at[q, b]], rows.at[b], sg[b])
      @pl.when((r < BLOCKS - 2) & (p == 0))
      def _ii0():
        issue_idx(r + 2, 0)
      @pl.when((r < BLOCKS - 2) & (p == 1))
      def _ii1():
        issue_idx(r + 2, 1)
      return _
    lax.fori_loop(0, BLOCKS, rnd, 0)

    plsc.subcore_barrier()
    obase = s * OUTROWS
    pltpu.sync_copy(acc.at[pl.ds(obase, OUTROWS)],
                    out_hbm.at[c, pl.ds(obase, OUTROWS)])

  return pl.kernel(
      body,
      out_type=jax.ShapeDtypeStruct((NC, NOUT, C), jnp.float32),
      mesh=mesh,
      scratch_types=scratch,
      compiler_params=pltpu.CompilerParams(use_tc_tiling_on_sc=False),
  )


def _seg_gather(z, src2, dst2):
  return _make_seg(gather=True)(z, src2, dst2)


def _seg_ones(z, src2, dst2):
  return _make_seg(gather=False)(z, src2, dst2)


def kernel(y_soft, edge_index, y_true, mask):
  src = edge_index[0]
  dst = edge_index[1]
  M = mask.shape[0]

  P = EPAD - E
  ar = jnp.arange(P, dtype=jnp.int32)
  pad_src = (ar * 97) % N
  pad_dst = N + (ar % NDUMP)
  src2 = jnp.concatenate([src, pad_src]).reshape(EPAD // CHUNK, CHUNK)
  dst2 = jnp.concatenate([dst, pad_dst]).reshape(EPAD // CHUNK, CHUNK)

  def seg(z):
    parts = _seg_gather(z, src2, dst2)
    return (parts[0] + parts[1])[:N]

  zdummy = jnp.zeros((N, C), jnp.float32)
  degp = _seg_ones(zdummy, src2, dst2)
  degs = jnp.clip((degp[0] + degp[1])[:N, 0], 1.0, None)
  norm = (degs ** -0.5)[:, None]

  y_true_oh = jax.nn.one_hot(y_true, C, dtype=y_soft.dtype)

  def label_prop(y, num_layers, alpha, lo, hi):
    last = (1.0 - alpha) * y
    for _ in range(num_layers):
      agg = seg(norm * y)
      y = jnp.clip(last + alpha * (agg * norm), lo, hi)
    return y

  err_top = y_true_oh - y_soft[:M]
  error = jnp.concatenate([err_top, jnp.zeros((N - M, C), y_soft.dtype)])
  smoothed = label_prop(error, NUM_CORRECTION_LAYERS, CORRECTION_ALPHA,
                        -1.0, 1.0)
  sigma = jnp.abs(err_top).sum() / M
  scale = sigma / jnp.abs(smoothed).sum(axis=1, keepdims=True)
  scale = jnp.where(jnp.isinf(scale) | (scale > 1000.0), 1.0, scale)
  result = y_soft + scale * smoothed
  result = jnp.where(jnp.isnan(result), y_soft, result)

  y = jnp.concatenate([y_true_oh, result[M:]])
  out = label_prop(y, NUM_SMOOTHING_LAYERS, SMOOTHING_ALPHA, 0.0, 1.0)
  return out


# speedup vs baseline: 72.2262x; 1.5657x over previous
import functools

import jax
import jax.numpy as jnp
from jax import lax
from jax.experimental import pallas as pl
from jax.experimental.pallas import tpu as pltpu
from jax.experimental.pallas import tpu_sc as plsc

N = 100000
E = 3200000
C = 16
NUM_CORRECTION_LAYERS = 10
CORRECTION_ALPHA = 0.979
NUM_SMOOTHING_LAYERS = 10
SMOOTHING_ALPHA = 0.756

NC = 2
NS = 16
NW = NC * NS

CHUNK = 128
CPB = 8
BLOCKS = 98
WCHUNKS = CPB * BLOCKS
WEDGES = WCHUNKS * CHUNK
EPAD = NW * WEDGES
NPAD = 100352
NDUMP = NPAD - N
ZROWS = NPAD // NS
ZCOPY = ZROWS // CHUNK
NOUT = 100096
OUTROWS = NOUT // NS
NBUF = 8
NSLOT = 12
PFULL = 48
PTAIL = OUTROWS - PFULL * CHUNK

_SCRATCH = (
    [pltpu.VMEM((2, CPB, CHUNK), jnp.int32),
     pltpu.VMEM((2, CPB, CHUNK), jnp.int32),
     pltpu.VMEM((NSLOT, CHUNK, C), jnp.float32),
     pltpu.VMEM_SHARED((NPAD, C), jnp.float32)]
    + [pltpu.SemaphoreType.DMA] * (2 * NBUF + 2)
)
_MESH = dict(core_axis_name="c", subcore_axis_name="s")
_PARAMS = pltpu.CompilerParams(use_tc_tiling_on_sc=False)


def _zero_acc(rows, acc, s, slot):
  def st(i, _):
    rows[slot, i, :] = jnp.zeros((C,), jnp.float32)
    return _
  lax.fori_loop(0, CHUNK, st, 0)
  zbase = s * ZROWS
  def zcp(i, _):
    pltpu.sync_copy(rows.at[slot], acc.at[pl.ds(zbase + i * CHUNK, CHUNK)])
    return _
  lax.fori_loop(0, ZCOPY, zcp, 0)


def _edge_pipeline(z_ref, src_hbm, dst_hbm, idx_src, idx_dst, rows, acc,
                   sg, ss, sis, wrow, gather):
  def issue_idx(blk, p):
    row = wrow + blk * CPB
    if gather:
      pltpu.async_copy(src_hbm.at[pl.ds(row, CPB)], idx_src.at[p], sis[p])
    pltpu.async_copy(dst_hbm.at[pl.ds(row, CPB)], idx_dst.at[p], sis[p])

  def wait_idx(p):
    if gather:
      pltpu.make_async_copy(src_hbm.at[pl.ds(0, CPB)], idx_src.at[p],
                            sis[p]).wait()
    pltpu.make_async_copy(dst_hbm.at[pl.ds(0, CPB)], idx_dst.at[p],
                          sis[p]).wait()

  issue_idx(0, 0)
  issue_idx(1, 1)
  wait_idx(0)
  if gather:
    for b in range(NBUF):
      pltpu.async_copy(z_ref.at[idx_src.at[0, b]], rows.at[b], sg[b])

  def rnd(r, _):
    p = r % 2
    q = 1 - p
    for b in range(NBUF):
      if gather:
        pltpu.make_async_copy(z_ref.at[idx_src.at[0, 0]], rows.at[b],
                              sg[b]).wait()
      pltpu.async_copy(rows.at[b], acc.at[idx_dst.at[p, b]], ss[b],
                       add=True)
    @pl.when((r < BLOCKS - 1) & (q == 0))
    def _wi0():
      wait_idx(0)
    @pl.when((r < BLOCKS - 1) & (q == 1))
    def _wi1():
      wait_idx(1)
    for b in range(NBUF):
      pltpu.make_async_copy(rows.at[b], acc.at[idx_dst.at[0, 0]],
                            ss[b]).wait()
      if gather:
        @pl.when(r < BLOCKS - 1)
        def _fg():
          pltpu.async_copy(z_ref.at[idx_src.at[q, b]], rows.at[b], sg[b])
    @pl.when((r < BLOCKS - 2) & (p == 0))
    def _ii0():
      issue_idx(r + 2, 0)
    @pl.when((r < BLOCKS - 2) & (p == 1))
    def _ii1():
      issue_idx(r + 2, 1)
    return _
  lax.fori_loop(0, BLOCKS, rnd, 0)


def _write_out(acc, out_hbm, c, s):
  obase = s * OUTROWS
  pltpu.sync_copy(acc.at[pl.ds(obase, OUTROWS)],
                  out_hbm.at[c, pl.ds(obase, OUTROWS)])


@functools.lru_cache(maxsize=None)
def _make_seg(gather: bool):
  def body(z_hbm, src_hbm, dst_hbm, out_hbm, idx_src, idx_dst, rows, acc,
           *sems):
    c = lax.axis_index("c")
    s = lax.axis_index("s")
    wrow = (s * NC + c) * WCHUNKS
    sg, ss, sis = sems[0:NBUF], sems[NBUF:2 * NBUF], sems[2 * NBUF:]

    _zero_acc(rows, acc, s, 0)
    if not gather:
      for b in range(NBUF):
        def st(i, _, b=b):
          rows[b, i, :] = jnp.ones((C,), jnp.float32)
          return _
        lax.fori_loop(0, CHUNK, st, 0)
    plsc.subcore_barrier()
    _edge_pipeline(z_hbm, src_hbm, dst_hbm, idx_src, idx_dst, rows, acc,
                   sg, ss, sis, wrow, gather)
    plsc.subcore_barrier()
    _write_out(acc, out_hbm, c, s)

  return pl.kernel(
      body,
      out_type=jax.ShapeDtypeStruct((NC, NOUT, C), jnp.float32),
      mesh=plsc.VectorSubcoreMesh(**_MESH),
      scratch_types=list(_SCRATCH),
      compiler_params=_PARAMS,
  )


@functools.lru_cache(maxsize=None)
def _make_fused(alpha: float, lo: float, hi: float):
  def body(p_hbm, last_hbm, anb_hbm, nb_hbm, src_hbm, dst_hbm,
           out_hbm, z_out, idx_src, idx_dst, rows, acc, *sems):
    c = lax.axis_index("c")
    s = lax.axis_index("s")
    wrow = (s * NC + c) * WCHUNKS
    sg, ss, sis = sems[0:NBUF], sems[NBUF:2 * NBUF], sems[2 * NBUF:]
    pb = s * OUTROWS

    _zero_acc(rows, acc, s, 11)

    def fire_in(st, k, r):
      sl = pl.ds(pb + k * CHUNK, r)
      pltpu.async_copy(p_hbm.at[0, sl], rows.at[6 * st + 0, pl.ds(0, r)],
                       sg[st])
      pltpu.async_copy(p_hbm.at[1, sl], rows.at[6 * st + 1, pl.ds(0, r)],
                       sg[st])
      pltpu.async_copy(last_hbm.at[sl], rows.at[6 * st + 2, pl.ds(0, r)],
                       sg[st])
      pltpu.async_copy(anb_hbm.at[sl], rows.at[6 * st + 3, pl.ds(0, r)],
                       sg[st])
      pltpu.async_copy(nb_hbm.at[sl], rows.at[6 * st + 4, pl.ds(0, r)],
                       sg[st])

    def drain_in(st, r):
      for j in range(5):
        pltpu.make_async_copy(last_hbm.at[pl.ds(0, r)],
                              rows.at[6 * st + j, pl.ds(0, r)],
                              sg[st]).wait()

    def compute(st, r):
      a0, a1, l, an, nv, zo = (6 * st + j for j in range(6))
      def row(i, _):
        t = rows[l, i, :] + rows[an, i, :] * (rows[a0, i, :] +
                                              rows[a1, i, :])
        y = jnp.clip(t, lo, hi)
        rows[zo, i, :] = rows[nv, i, :] * y
        return _
      lax.fori_loop(0, r, row, 0)

    def fire_wb(st, k, r):
      pltpu.async_copy(rows.at[6 * st + 5, pl.ds(0, r)],
                       z_out.at[pl.ds(pb + k * CHUNK, r)], ss[st])

    def wait_wb(st, r=CHUNK):
      pltpu.make_async_copy(rows.at[6 * st + 5, pl.ds(0, r)],
                            z_out.at[pl.ds(0, r)], ss[st]).wait()

    fire_in(0, 0, CHUNK)
    def pair(i, _):
      k0 = i * 2
      fire_in(1, k0 + 1, CHUNK)
      @pl.when(i > 0)
      def _wa():
        wait_wb(0)
      drain_in(0, CHUNK)
      compute(0, CHUNK)
      fire_wb(0, k0, CHUNK)
      @pl.when(i < PFULL // 2 - 1)
      def _fa():
        fire_in(0, k0 + 2, CHUNK)
      @pl.when(i > 0)
      def _wb():
        wait_wb(1)
      drain_in(1, CHUNK)
      compute(1, CHUNK)
      fire_wb(1, k0 + 1, CHUNK)
      return _
    lax.fori_loop(0, PFULL // 2, pair, 0)
    wait_wb(0)
    fire_in(0, PFULL, PTAIL)
    drain_in(0, PTAIL)
    compute(0, PTAIL)
    fire_wb(0, PFULL, PTAIL)
    wait_wb(0, PTAIL)
    wait_wb(1)

    plsc.subcore_barrier()
    _edge_pipeline(z_out, src_hbm, dst_hbm, idx_src, idx_dst, rows, acc,
                   sg, ss, sis, wrow, True)
    plsc.subcore_barrier()
    _write_out(acc, out_hbm, c, s)

  return pl.kernel(
      body,
      out_type=(jax.ShapeDtypeStruct((NC, NOUT, C), jnp.float32),
                jax.ShapeDtypeStruct((NOUT, C), jnp.float32)),
      mesh=plsc.VectorSubcoreMesh(**_MESH),
      scratch_types=list(_SCRATCH),
      compiler_params=_PARAMS,
  )


def _seg_gather(z, src2, dst2):
  return _make_seg(True)(z, src2, dst2)


def _seg_ones(z, src2, dst2):
  return _make_seg(False)(z, src2, dst2)


def kernel(y_soft, edge_index, y_true, mask):
  src = edge_index[0]
  dst = edge_index[1]
  M = mask.shape[0]

  P = EPAD - E
  ar = jnp.arange(P, dtype=jnp.int32)
  pad_src = (ar * 97) % N
  pad_dst = N + (ar % NDUMP)
  src2 = jnp.concatenate([src, pad_src]).reshape(EPAD // CHUNK, CHUNK)
  dst2 = jnp.concatenate([dst, pad_dst]).reshape(EPAD // CHUNK, CHUNK)

  zdummy = jnp.zeros((N, C), jnp.float32)
  degp = _seg_ones(zdummy, src2, dst2)
  degs = jnp.clip((degp[0] + degp[1])[:N, 0], 1.0, None)
  norm1 = degs ** -0.5
  norm = norm1[:, None]
  norm_out = jnp.concatenate([norm1, jnp.ones((NOUT - N,), jnp.float32)])
  nb = jnp.broadcast_to(norm_out[:, None], (NOUT, C))

  y_true_oh = jax.nn.one_hot(y_true, C, dtype=y_soft.dtype)

  def label_prop(y, num_layers, alpha, lo, hi):
    last = (1.0 - alpha) * y
    last_pad = jnp.concatenate(
        [last, jnp.zeros((NOUT - N, C), jnp.float32)])
    anb = alpha * nb
    parts = _seg_gather(norm * y, src2, dst2)
    fused = _make_fused(alpha, lo, hi)
    for _ in range(num_layers - 1):
      parts, _ = fused(parts, last_pad, anb, nb, src2, dst2)
    agg = (parts[0] + parts[1])[:N]
    return jnp.clip(last + alpha * (agg * norm), lo, hi)

  err_top = y_true_oh - y_soft[:M]
  error = jnp.concatenate([err_top, jnp.zeros((N - M, C), y_soft.dtype)])
  smoothed = label_prop(error, NUM_CORRECTION_LAYERS, CORRECTION_ALPHA,
                        -1.0, 1.0)
  sigma = jnp.abs(err_top).sum() / M
  scale = sigma / jnp.abs(smoothed).sum(axis=1, keepdims=True)
  scale = jnp.where(jnp.isinf(scale) | (scale > 1000.0), 1.0, scale)
  result = y_soft + scale * smoothed
  result = jnp.where(jnp.isnan(result), y_soft, result)

  y = jnp.concatenate([y_true_oh, result[M:]])
  out = label_prop(y, NUM_SMOOTHING_LAYERS, SMOOTHING_ALPHA, 0.0, 1.0)
  return out
